# baseline (speedup 1.0000x reference)
import jax
import jax.numpy as jnp
from jax.experimental import pallas as pl
from jax.experimental.pallas import tpu as pltpu

INC = 128
INN = 512
OUTC = 64
OUTN = 64
MAXD = 8
N = 512

BH = 32
HALF = N // 2
STEPS = HALF // BH


def _body(a_ref, m_ref, w_ref, b_ref, xa_ref, xb_ref, o_ref, p_scr):
    @pl.when(pl.program_id(0) == 0)
    def _():
        rows = jax.lax.broadcasted_iota(jnp.int32, (INN, OUTN), 0)
        acc = jnp.zeros((INN, OUTN), jnp.float32)
        for d in range(MAXD):
            acc = acc + jnp.where(rows == a_ref[d : d + 1, :],
                                  m_ref[d : d + 1, :], 0.0)
        p_scr[...] = acc.astype(jnp.bfloat16)

    for h, x_ref in ((0, xa_ref), (1, xb_ref)):
        xb = x_ref[...].reshape(BH * INC, INN).astype(jnp.bfloat16)
        xp = jnp.dot(xb, p_scr[...], preferred_element_type=jnp.float32)
        xp = xp.reshape(BH, INC, OUTN)
        yb = jax.lax.dot_general(xp, w_ref[...], (((1,), (0,)), ((), ())),
                                 preferred_element_type=jnp.float32)
        o_ref[h] = jnp.transpose(yb, (0, 2, 1)) + b_ref[...][None, :, :]


@jax.jit
def kernel(x, A, mask, weight, bias):
    at = A.T.astype(jnp.int32)
    mt = mask[:, :, 0].T
    out = pl.pallas_call(
        _body,
        grid=(STEPS,),
        in_specs=[
            pl.BlockSpec((MAXD, OUTN), lambda i: (0, 0)),
            pl.BlockSpec((MAXD, OUTN), lambda i: (0, 0)),
            pl.BlockSpec((INC, OUTC), lambda i: (0, 0)),
            pl.BlockSpec((OUTC, 1), lambda i: (0, 0)),
            pl.BlockSpec((BH, INC, INN), lambda i: (i, 0, 0)),
            pl.BlockSpec((BH, INC, INN), lambda i: (i + STEPS, 0, 0)),
        ],
        out_specs=pl.BlockSpec((2, BH, OUTC, OUTN), lambda i: (0, i, 0, 0)),
        out_shape=jax.ShapeDtypeStruct((2, HALF, OUTC, OUTN), jnp.float32),
        scratch_shapes=[pltpu.VMEM((INN, OUTN), jnp.bfloat16)],
    )(at, mt, weight, bias, x, x)
    return out.reshape(N, OUTC, OUTN)
